# baseline (speedup 1.0000x reference)
import functools

import jax
import jax.numpy as jnp
from jax import lax
from jax.experimental import pallas as pl
from jax.experimental.pallas import tpu as pltpu
from jax.experimental.pallas import tpu_sc as plsc

S, B, C, H, W = 2, 4, 1, 152, 272
M = 256
HW = H * W
CHUNK = 64
NSTREAM = 11


def _sc_gather_body(ind_hbm, hm_hbm, reg_hbm, wh_hbm, co_hbm, box_hbm,
                    hmg, regg, whg, cog, boxg,
                    idx_v, off_v, val_v, sem):
    core = lax.axis_index("c")
    sub = lax.axis_index("s")
    wid = sub * 2 + core
    s = wid // 16
    rem = wid - s * 16
    b = rem // 4
    mb = (rem - b * 4) * CHUNK
    base = pl.multiple_of(b * M + mb, CHUNK)
    pltpu.sync_copy(ind_hbm.at[pl.ds(base, CHUNK)], idx_v)

    sb = s * B + b

    def lin(off):
        return lambda v: v + off

    def box_off(cc):
        off = sb * (HW * 4) + cc * 128
        return lambda v: ((v >> 7) << 9) + (v & 127) + off

    streams = [(hm_hbm, lin(sb * HW), hmg.at[s, b, pl.ds(mb, CHUNK)])]
    for tab, out in ((reg_hbm, regg), (wh_hbm, whg), (co_hbm, cog)):
        for cc in range(2):
            streams.append((tab, lin((sb * 2 + cc) * HW),
                            out.at[s, b, cc, pl.ds(mb, CHUNK)]))
    for cc in range(4):
        streams.append((box_hbm, box_off(cc),
                        boxg.at[s, b, cc, pl.ds(mb, CHUNK)]))

    for j, (_, fn, _) in enumerate(streams):
        for k in range(CHUNK // 16):
            off_v[j, pl.ds(k * 16, 16)] = fn(idx_v[pl.ds(k * 16, 16)])
    descs = [pltpu.async_copy(tab.at[off_v.at[j]], val_v.at[j], sem)
             for j, (tab, _, _) in enumerate(streams)]
    for d in descs:
        d.wait()
    for j, (_, _, dst) in enumerate(streams):
        pltpu.sync_copy(val_v.at[j], dst)


@functools.cache
def _get_sc_gather():
    return pl.kernel(
        _sc_gather_body,
        out_type=(
            jax.ShapeDtypeStruct((S, B, M), jnp.float32),
            jax.ShapeDtypeStruct((S, B, 2, M), jnp.float32),
            jax.ShapeDtypeStruct((S, B, 2, M), jnp.float32),
            jax.ShapeDtypeStruct((S, B, 2, M), jnp.float32),
            jax.ShapeDtypeStruct((S, B, 4, M), jnp.float32),
        ),
        mesh=plsc.VectorSubcoreMesh(core_axis_name="c", subcore_axis_name="s"),
        scratch_types=[
            pltpu.VMEM((CHUNK,), jnp.int32),
            pltpu.VMEM((NSTREAM, CHUNK), jnp.int32),
            pltpu.VMEM((NSTREAM, CHUNK), jnp.float32),
            pltpu.SemaphoreType.DMA,
        ],
    )


def _clip_sig(x):
    return jnp.clip(1.0 / (1.0 + jnp.exp(-x)), 1e-4, 1.0 - 1e-4)


def _tc_dense_body(hm_ref, hmr_ref, tr_ref, trr_ref, trm_ref, out_ref):
    trm = trm_ref[...]
    trr = trr_ref[...]
    hmr = hmr_ref[...]
    omt = 1.0 - hmr
    omt2 = omt * omt
    omt4 = omt2 * omt2
    vals = []
    for s in range(S):
        hs = _clip_sig(hm_ref[s])
        vals.append(jnp.sum(jnp.log(1.0 - hs) * hs * hs * omt4))
    for s in range(S):
        vals.append(jnp.sum(jnp.abs(tr_ref[s] * trm - trr * trm)))
    vals.append(jnp.sum(trm))

    r_i = lax.broadcasted_iota(jnp.int32, (8, 128), 0)
    c_i = lax.broadcasted_iota(jnp.int32, (8, 128), 1)
    acc = jnp.zeros((8, 128), jnp.float32)
    for k, v in enumerate(vals):
        acc = acc + jnp.where((r_i == 0) & (c_i == k), v, 0.0)
    out_ref[...] = acc


def _tc_combine_body(part_ref, hmg_ref, mask_ref, regp_ref, regt_ref,
                     regm_ref, whp_ref, wht_ref, whm_ref, cop_ref, cot_ref,
                     com_ref, boxp_ref, boxt_ref, boxm_ref, out_ref):
    c_i1 = lax.broadcasted_iota(jnp.int32, (1, 128), 1)
    part = part_ref[0:1, :]

    def pget(k):
        return jnp.sum(jnp.where(c_i1 == k, part, 0.0))

    num_pos = jnp.sum(mask_ref[...])
    npos_den = jnp.maximum(num_pos, 1.0)
    tr_den = pget(4) + 1e-4

    def masked_l1(pred, tgt, m):
        return jnp.sum(jnp.abs(pred * m - tgt * m)) / (jnp.sum(m) + 1e-4)

    boxm = boxm_ref[...]
    nb = jnp.sum(boxm) + 1e-4

    def xyxy(cx, cy, w, h):
        return cx - 0.5 * w, cy - 0.5 * h, cx + 0.5 * w, cy + 0.5 * h

    tcoord = [boxt_ref[:, c, :] for c in range(4)]
    x21, y21, x22, y22 = xyxy(*tcoord)
    area2 = (x22 - x21) * (y22 - y21)

    losses = []
    for s in range(S):
        neg = pget(s)
        pv = _clip_sig(hmg_ref[s])
        one_m_pv = 1.0 - pv
        pos = jnp.sum(jnp.log(pv) * one_m_pv * one_m_pv * mask_ref[...])
        focal = jnp.where(num_pos == 0.0, -neg, -(pos + neg) / npos_den)
        track = pget(2 + s) / tr_den
        regl = masked_l1(regp_ref[s], regt_ref[...], regm_ref[...])
        whl = masked_l1(whp_ref[s], wht_ref[...], whm_ref[...])
        col = masked_l1(cop_ref[s], cot_ref[...], com_ref[...])
        pcoord = [boxp_ref[s, :, c, :] for c in range(4)]
        l1 = sum(jnp.sum(jnp.abs(p - t) * boxm)
                 for p, t in zip(pcoord, tcoord)) / nb
        x11, y11, x12, y12 = xyxy(*pcoord)
        area1 = (x12 - x11) * (y12 - y11)
        iw = jnp.maximum(jnp.minimum(x12, x22) - jnp.maximum(x11, x21), 0.0)
        ih = jnp.maximum(jnp.minimum(y12, y22) - jnp.maximum(y11, y21), 0.0)
        inter = iw * ih
        union = area1 + area2 - inter
        iou = inter / (union + 1e-7)
        cw = jnp.maximum(jnp.maximum(x12, x22) - jnp.minimum(x11, x21), 0.0)
        ch = jnp.maximum(jnp.maximum(y12, y22) - jnp.minimum(y11, y21), 0.0)
        areac = cw * ch
        giou = iou - (areac - union) / (areac + 1e-7)
        gl = jnp.sum((1.0 - giou) * boxm) / nb
        losses += [focal, track, regl, whl, col, l1, gl]

    r_i = lax.broadcasted_iota(jnp.int32, (8, 128), 0)
    c_i = lax.broadcasted_iota(jnp.int32, (8, 128), 1)
    acc = jnp.zeros((8, 128), jnp.float32)
    for k, v in enumerate(losses):
        acc = acc + jnp.where((r_i == 0) & (c_i == k), v, 0.0)
    out_ref[...] = acc


def _tc_dense(*args):
    return pl.pallas_call(
        _tc_dense_body,
        out_shape=jax.ShapeDtypeStruct((8, 128), jnp.float32),
    )(*args)


def _tc_combine(*args):
    return pl.pallas_call(
        _tc_combine_body,
        out_shape=jax.ShapeDtypeStruct((8, 128), jnp.float32),
    )(*args)


def kernel(hm, reg, wh, center_offset, tracking, boxes, hm_r, reg_r, wh_r,
           center_offset_r, tracking_r, tracking_mask_r, mask_r, reg_mask_r,
           wh_mask_r, center_offset_mask_r, boxes_r, boxes_mask_r, ind_r,
           cat_r):
    del cat_r
    hm_g, reg_g, wh_g, co_g, box_g = _get_sc_gather()(
        ind_r.reshape(B * M),
        hm.reshape(S * B * HW),
        reg.reshape(S * B * 2 * HW),
        wh.reshape(S * B * 2 * HW),
        center_offset.reshape(S * B * 2 * HW),
        boxes.reshape(S, B, HW // 128, 128, 4)
        .transpose(0, 1, 2, 4, 3).reshape(S * B * HW * 4),
    )
    part = _tc_dense(
        hm.reshape(S, B, H, W),
        hm_r.reshape(B, H, W),
        tracking.reshape(S, B * 2, H, W),
        tracking_r.reshape(B * 2, H, W),
        tracking_mask_r.reshape(B * 2, H, W),
    )
    out = _tc_combine(
        part,
        hm_g,
        mask_r,
        reg_g,
        reg_r.transpose(0, 2, 1),
        reg_mask_r.transpose(0, 2, 1),
        wh_g,
        wh_r.transpose(0, 2, 1),
        wh_mask_r.transpose(0, 2, 1),
        co_g,
        center_offset_r.transpose(0, 2, 1),
        center_offset_mask_r.transpose(0, 2, 1),
        box_g,
        boxes_r.transpose(0, 2, 1),
        boxes_mask_r,
    )
    return out[0, :14]


# speedup vs baseline: 1.0645x; 1.0645x over previous
import functools

import jax
import jax.numpy as jnp
from jax import lax
from jax.experimental import pallas as pl
from jax.experimental.pallas import tpu as pltpu
from jax.experimental.pallas import tpu_sc as plsc

S, B, C, H, W = 2, 4, 1, 152, 272
M = 256
HW = H * W
CHUNK = 64
NSTREAM = 11


def _sc_gather_body(ind_hbm, hm_hbm, reg_hbm, wh_hbm, co_hbm, box_hbm,
                    hmg, regg, whg, cog, boxg,
                    idx_v, off_v, val_v, sem):
    core = lax.axis_index("c")
    sub = lax.axis_index("s")
    wid = sub * 2 + core
    s = wid // 16
    rem = wid - s * 16
    b = rem // 4
    mb = (rem - b * 4) * CHUNK
    base = pl.multiple_of(b * M + mb, CHUNK)
    pltpu.sync_copy(ind_hbm.at[pl.ds(base, CHUNK)], idx_v)

    sb = s * B + b

    def lin(off):
        return lambda v: v + off

    def box_off(cc):
        return lin((sb * 4 + cc) * HW)

    streams = [(hm_hbm, lin(sb * HW), hmg.at[s, b, pl.ds(mb, CHUNK)])]
    for tab, out in ((reg_hbm, regg), (wh_hbm, whg), (co_hbm, cog)):
        for cc in range(2):
            streams.append((tab, lin((sb * 2 + cc) * HW),
                            out.at[s, b, cc, pl.ds(mb, CHUNK)]))
    for cc in range(4):
        streams.append((box_hbm, box_off(cc),
                        boxg.at[s, b, cc, pl.ds(mb, CHUNK)]))

    for j, (_, fn, _) in enumerate(streams):
        for k in range(CHUNK // 16):
            off_v[j, pl.ds(k * 16, 16)] = fn(idx_v[pl.ds(k * 16, 16)])
    descs = [pltpu.async_copy(tab.at[off_v.at[j]], val_v.at[j], sem)
             for j, (tab, _, _) in enumerate(streams)]
    for d in descs:
        d.wait()
    for j, (_, _, dst) in enumerate(streams):
        pltpu.sync_copy(val_v.at[j], dst)


@functools.cache
def _get_sc_gather():
    return pl.kernel(
        _sc_gather_body,
        out_type=(
            jax.ShapeDtypeStruct((S, B, M), jnp.float32),
            jax.ShapeDtypeStruct((S, B, 2, M), jnp.float32),
            jax.ShapeDtypeStruct((S, B, 2, M), jnp.float32),
            jax.ShapeDtypeStruct((S, B, 2, M), jnp.float32),
            jax.ShapeDtypeStruct((S, B, 4, M), jnp.float32),
        ),
        mesh=plsc.VectorSubcoreMesh(core_axis_name="c", subcore_axis_name="s"),
        scratch_types=[
            pltpu.VMEM((CHUNK,), jnp.int32),
            pltpu.VMEM((NSTREAM, CHUNK), jnp.int32),
            pltpu.VMEM((NSTREAM, CHUNK), jnp.float32),
            pltpu.SemaphoreType.DMA,
        ],
    )


def _clip_sig(x):
    return jnp.clip(1.0 / (1.0 + jnp.exp(-x)), 1e-4, 1.0 - 1e-4)


def _tc_dense_body(hm_ref, hmr_ref, tr_ref, trr_ref, trm_ref, out_ref):
    trm = trm_ref[...]
    trr = trr_ref[...]
    hmr = hmr_ref[...]
    omt = 1.0 - hmr
    omt2 = omt * omt
    omt4 = omt2 * omt2
    vals = []
    for s in range(S):
        hs = _clip_sig(hm_ref[s])
        vals.append(jnp.sum(jnp.log(1.0 - hs) * hs * hs * omt4))
    for s in range(S):
        vals.append(jnp.sum(jnp.abs(tr_ref[s] * trm - trr * trm)))
    vals.append(jnp.sum(trm))

    r_i = lax.broadcasted_iota(jnp.int32, (8, 128), 0)
    c_i = lax.broadcasted_iota(jnp.int32, (8, 128), 1)
    acc = jnp.zeros((8, 128), jnp.float32)
    for k, v in enumerate(vals):
        acc = acc + jnp.where((r_i == 0) & (c_i == k), v, 0.0)
    out_ref[...] = acc


def _tc_combine_body(part_ref, hmg_ref, mask_ref, regp_ref, regt_ref,
                     regm_ref, whp_ref, wht_ref, whm_ref, cop_ref, cot_ref,
                     com_ref, boxp_ref, boxt_ref, boxm_ref, out_ref):
    c_i1 = lax.broadcasted_iota(jnp.int32, (1, 128), 1)
    part = part_ref[0:1, :]

    def pget(k):
        return jnp.sum(jnp.where(c_i1 == k, part, 0.0))

    num_pos = jnp.sum(mask_ref[...])
    npos_den = jnp.maximum(num_pos, 1.0)
    tr_den = pget(4) + 1e-4

    def masked_l1(pred, tgt, m):
        return jnp.sum(jnp.abs(pred * m - tgt * m)) / (jnp.sum(m) + 1e-4)

    boxm = boxm_ref[...]
    nb = jnp.sum(boxm) + 1e-4

    def xyxy(cx, cy, w, h):
        return cx - 0.5 * w, cy - 0.5 * h, cx + 0.5 * w, cy + 0.5 * h

    tcoord = [boxt_ref[:, c, :] for c in range(4)]
    x21, y21, x22, y22 = xyxy(*tcoord)
    area2 = (x22 - x21) * (y22 - y21)

    losses = []
    for s in range(S):
        neg = pget(s)
        pv = _clip_sig(hmg_ref[s])
        one_m_pv = 1.0 - pv
        pos = jnp.sum(jnp.log(pv) * one_m_pv * one_m_pv * mask_ref[...])
        focal = jnp.where(num_pos == 0.0, -neg, -(pos + neg) / npos_den)
        track = pget(2 + s) / tr_den
        regl = masked_l1(regp_ref[s], regt_ref[...], regm_ref[...])
        whl = masked_l1(whp_ref[s], wht_ref[...], whm_ref[...])
        col = masked_l1(cop_ref[s], cot_ref[...], com_ref[...])
        pcoord = [boxp_ref[s, :, c, :] for c in range(4)]
        l1 = sum(jnp.sum(jnp.abs(p - t) * boxm)
                 for p, t in zip(pcoord, tcoord)) / nb
        x11, y11, x12, y12 = xyxy(*pcoord)
        area1 = (x12 - x11) * (y12 - y11)
        iw = jnp.maximum(jnp.minimum(x12, x22) - jnp.maximum(x11, x21), 0.0)
        ih = jnp.maximum(jnp.minimum(y12, y22) - jnp.maximum(y11, y21), 0.0)
        inter = iw * ih
        union = area1 + area2 - inter
        iou = inter / (union + 1e-7)
        cw = jnp.maximum(jnp.maximum(x12, x22) - jnp.minimum(x11, x21), 0.0)
        ch = jnp.maximum(jnp.maximum(y12, y22) - jnp.minimum(y11, y21), 0.0)
        areac = cw * ch
        giou = iou - (areac - union) / (areac + 1e-7)
        gl = jnp.sum((1.0 - giou) * boxm) / nb
        losses += [focal, track, regl, whl, col, l1, gl]

    r_i = lax.broadcasted_iota(jnp.int32, (8, 128), 0)
    c_i = lax.broadcasted_iota(jnp.int32, (8, 128), 1)
    acc = jnp.zeros((8, 128), jnp.float32)
    for k, v in enumerate(losses):
        acc = acc + jnp.where((r_i == 0) & (c_i == k), v, 0.0)
    out_ref[...] = acc


def _tc_dense(*args):
    return pl.pallas_call(
        _tc_dense_body,
        out_shape=jax.ShapeDtypeStruct((8, 128), jnp.float32),
    )(*args)


def _tc_combine(*args):
    return pl.pallas_call(
        _tc_combine_body,
        out_shape=jax.ShapeDtypeStruct((8, 128), jnp.float32),
    )(*args)


def kernel(hm, reg, wh, center_offset, tracking, boxes, hm_r, reg_r, wh_r,
           center_offset_r, tracking_r, tracking_mask_r, mask_r, reg_mask_r,
           wh_mask_r, center_offset_mask_r, boxes_r, boxes_mask_r, ind_r,
           cat_r):
    del cat_r
    hm_g, reg_g, wh_g, co_g, box_g = _get_sc_gather()(
        ind_r.reshape(B * M),
        hm.reshape(S * B * HW),
        reg.reshape(S * B * 2 * HW),
        wh.reshape(S * B * 2 * HW),
        center_offset.reshape(S * B * 2 * HW),
        boxes.transpose(0, 1, 3, 2).reshape(S * B * 4 * HW),
    )
    part = _tc_dense(
        hm.reshape(S, B, H, W),
        hm_r.reshape(B, H, W),
        tracking.reshape(S, B * 2, H, W),
        tracking_r.reshape(B * 2, H, W),
        tracking_mask_r.reshape(B * 2, H, W),
    )
    out = _tc_combine(
        part,
        hm_g,
        mask_r,
        reg_g,
        reg_r.transpose(0, 2, 1),
        reg_mask_r.transpose(0, 2, 1),
        wh_g,
        wh_r.transpose(0, 2, 1),
        wh_mask_r.transpose(0, 2, 1),
        co_g,
        center_offset_r.transpose(0, 2, 1),
        center_offset_mask_r.transpose(0, 2, 1),
        box_g,
        boxes_r.transpose(0, 2, 1),
        boxes_mask_r,
    )
    return out[0, :14]


# speedup vs baseline: 1.0909x; 1.0247x over previous
import functools

import jax
import jax.numpy as jnp
from jax import lax
from jax.experimental import pallas as pl
from jax.experimental.pallas import tpu as pltpu
from jax.experimental.pallas import tpu_sc as plsc

S, B, C, H, W = 2, 4, 1, 152, 272
M = 256
HW = H * W
CHUNK = 64
NSTREAM = 11


def _sc_gather_body(ind_hbm, hm_hbm, reg_hbm, wh_hbm, co_hbm, box_hbm,
                    hmg, regg, whg, cog, boxg,
                    idx_v, off_v, val_v, sem):
    core = lax.axis_index("c")
    sub = lax.axis_index("s")
    wid = sub * 2 + core
    s = wid // 16
    rem = wid - s * 16
    b = rem // 4
    mb = (rem - b * 4) * CHUNK
    base = pl.multiple_of(b * M + mb, CHUNK)
    pltpu.sync_copy(ind_hbm.at[pl.ds(base, CHUNK)], idx_v)

    sb = s * B + b

    def lin(off):
        return lambda v: v + off

    def box_off(cc):
        return lin((sb * 4 + cc) * HW)

    streams = [(hm_hbm, lin(sb * HW), hmg.at[s, b, pl.ds(mb, CHUNK)])]
    for tab, out in ((reg_hbm, regg), (wh_hbm, whg), (co_hbm, cog)):
        for cc in range(2):
            streams.append((tab, lin((sb * 2 + cc) * HW),
                            out.at[s, b, cc, pl.ds(mb, CHUNK)]))
    for cc in range(4):
        streams.append((box_hbm, box_off(cc),
                        boxg.at[s, b, cc, pl.ds(mb, CHUNK)]))

    for j, (_, fn, _) in enumerate(streams):
        for k in range(CHUNK // 16):
            off_v[j, pl.ds(k * 16, 16)] = fn(idx_v[pl.ds(k * 16, 16)])
    descs = [pltpu.async_copy(tab.at[off_v.at[j]], val_v.at[j], sem)
             for j, (tab, _, _) in enumerate(streams)]
    for d in descs:
        d.wait()
    for j, (_, _, dst) in enumerate(streams):
        pltpu.sync_copy(val_v.at[j], dst)


@functools.cache
def _get_sc_gather():
    return pl.kernel(
        _sc_gather_body,
        out_type=(
            jax.ShapeDtypeStruct((S, B, M), jnp.float32),
            jax.ShapeDtypeStruct((S, B, 2, M), jnp.float32),
            jax.ShapeDtypeStruct((S, B, 2, M), jnp.float32),
            jax.ShapeDtypeStruct((S, B, 2, M), jnp.float32),
            jax.ShapeDtypeStruct((S, B, 4, M), jnp.float32),
        ),
        mesh=plsc.VectorSubcoreMesh(core_axis_name="c", subcore_axis_name="s"),
        scratch_types=[
            pltpu.VMEM((CHUNK,), jnp.int32),
            pltpu.VMEM((NSTREAM, CHUNK), jnp.int32),
            pltpu.VMEM((NSTREAM, CHUNK), jnp.float32),
            pltpu.SemaphoreType.DMA,
        ],
    )


def _clip_sig(x):
    return jnp.clip(1.0 / (1.0 + jnp.exp(-x)), 1e-4, 1.0 - 1e-4)


def _tc_dense_body(hm_ref, hmr_ref, tr_ref, trr_ref, trm_ref, out_ref):
    trm = trm_ref[...]
    trr = trr_ref[...]
    hmr = hmr_ref[...]
    omt = 1.0 - hmr
    omt2 = omt * omt
    omt4 = omt2 * omt2
    vals = []
    for s in range(S):
        hs = _clip_sig(hm_ref[s])
        vals.append(jnp.sum(jnp.log(1.0 - hs) * hs * hs * omt4))
    for s in range(S):
        vals.append(jnp.sum(jnp.abs(tr_ref[s] * trm - trr * trm)))
    vals.append(jnp.sum(trm))

    r_i = lax.broadcasted_iota(jnp.int32, (8, 128), 0)
    c_i = lax.broadcasted_iota(jnp.int32, (8, 128), 1)
    acc = jnp.zeros((8, 128), jnp.float32)
    for k, v in enumerate(vals):
        acc = acc + jnp.where((r_i == 0) & (c_i == k), v, 0.0)
    out_ref[...] = acc


def _tc_combine_body(part_ref, hmg_ref, mask_ref, regp_ref, regt_ref,
                     regm_ref, whp_ref, wht_ref, whm_ref, cop_ref, cot_ref,
                     com_ref, boxp_ref, boxt_ref, boxm_ref, out_ref):
    c_i1 = lax.broadcasted_iota(jnp.int32, (1, 128), 1)
    part = part_ref[0:1, :]

    def pget(k):
        return jnp.sum(jnp.where(c_i1 == k, part, 0.0))

    num_pos = jnp.sum(mask_ref[...])
    npos_den = jnp.maximum(num_pos, 1.0)
    tr_den = pget(4) + 1e-4

    def masked_l1(pred, tgt, m):
        return jnp.sum(jnp.abs(pred * m - tgt * m)) / (jnp.sum(m) + 1e-4)

    boxm = boxm_ref[...]
    nb = jnp.sum(boxm) + 1e-4

    def xyxy(cx, cy, w, h):
        return cx - 0.5 * w, cy - 0.5 * h, cx + 0.5 * w, cy + 0.5 * h

    tcoord = [boxt_ref[:, c, :] for c in range(4)]
    x21, y21, x22, y22 = xyxy(*tcoord)
    area2 = (x22 - x21) * (y22 - y21)

    losses = []
    for s in range(S):
        neg = pget(s)
        pv = _clip_sig(hmg_ref[s])
        one_m_pv = 1.0 - pv
        pos = jnp.sum(jnp.log(pv) * one_m_pv * one_m_pv * mask_ref[...])
        focal = jnp.where(num_pos == 0.0, -neg, -(pos + neg) / npos_den)
        track = pget(2 + s) / tr_den
        regl = masked_l1(regp_ref[s], regt_ref[...], regm_ref[...])
        whl = masked_l1(whp_ref[s], wht_ref[...], whm_ref[...])
        col = masked_l1(cop_ref[s], cot_ref[...], com_ref[...])
        pcoord = [boxp_ref[s, :, c, :] for c in range(4)]
        l1 = sum(jnp.sum(jnp.abs(p - t) * boxm)
                 for p, t in zip(pcoord, tcoord)) / nb
        x11, y11, x12, y12 = xyxy(*pcoord)
        area1 = (x12 - x11) * (y12 - y11)
        iw = jnp.maximum(jnp.minimum(x12, x22) - jnp.maximum(x11, x21), 0.0)
        ih = jnp.maximum(jnp.minimum(y12, y22) - jnp.maximum(y11, y21), 0.0)
        inter = iw * ih
        union = area1 + area2 - inter
        iou = inter / (union + 1e-7)
        cw = jnp.maximum(jnp.maximum(x12, x22) - jnp.minimum(x11, x21), 0.0)
        ch = jnp.maximum(jnp.maximum(y12, y22) - jnp.minimum(y11, y21), 0.0)
        areac = cw * ch
        giou = iou - (areac - union) / (areac + 1e-7)
        gl = jnp.sum((1.0 - giou) * boxm) / nb
        losses += [focal, track, regl, whl, col, l1, gl]

    c_i = lax.broadcasted_iota(jnp.int32, (14,), 0)
    acc = jnp.zeros((14,), jnp.float32)
    for k, v in enumerate(losses):
        acc = acc + jnp.where(c_i == k, v, 0.0)
    out_ref[...] = acc


def _tc_dense(*args):
    return pl.pallas_call(
        _tc_dense_body,
        out_shape=jax.ShapeDtypeStruct((8, 128), jnp.float32),
    )(*args)


def _tc_combine(*args):
    return pl.pallas_call(
        _tc_combine_body,
        out_shape=jax.ShapeDtypeStruct((14,), jnp.float32),
    )(*args)


def kernel(hm, reg, wh, center_offset, tracking, boxes, hm_r, reg_r, wh_r,
           center_offset_r, tracking_r, tracking_mask_r, mask_r, reg_mask_r,
           wh_mask_r, center_offset_mask_r, boxes_r, boxes_mask_r, ind_r,
           cat_r):
    del cat_r
    hm_g, reg_g, wh_g, co_g, box_g = _get_sc_gather()(
        ind_r.reshape(B * M),
        hm.reshape(S * B * HW),
        reg.reshape(S * B * 2 * HW),
        wh.reshape(S * B * 2 * HW),
        center_offset.reshape(S * B * 2 * HW),
        boxes.transpose(0, 1, 3, 2).reshape(S * B * 4 * HW),
    )
    part = _tc_dense(
        hm.reshape(S, B, H, W),
        hm_r.reshape(B, H, W),
        tracking.reshape(S, B * 2, H, W),
        tracking_r.reshape(B * 2, H, W),
        tracking_mask_r.reshape(B * 2, H, W),
    )
    out = _tc_combine(
        part,
        hm_g,
        mask_r,
        reg_g,
        reg_r.transpose(0, 2, 1),
        reg_mask_r.transpose(0, 2, 1),
        wh_g,
        wh_r.transpose(0, 2, 1),
        wh_mask_r.transpose(0, 2, 1),
        co_g,
        center_offset_r.transpose(0, 2, 1),
        center_offset_mask_r.transpose(0, 2, 1),
        box_g,
        boxes_r.transpose(0, 2, 1),
        boxes_mask_r,
    )
    return out
